# baseline (speedup 1.0000x reference)
import functools

import jax
import jax.numpy as jnp
from jax import lax
from jax.experimental import pallas as pl
from jax.experimental.pallas import tpu as pltpu
from jax.experimental.pallas import tpu_sc as plsc

N = 10000
E = 160000
D = 256
G = 64
F1 = 16

NC = 2
NS = 16
CHUNK = 128
NCHUNK = 80
E_TILE = NCHUNK * CHUNK
E_PAD = E_TILE * NS
N_PAD = 10240
RPT = N_PAD // NS
NB = 10
BLK = N_PAD // NB
NRING = 8


def _rsqrt_newton(x):
    i = plsc.bitcast(x, jnp.int32)
    y = plsc.bitcast(jnp.int32(0x5F3759DF) - (i >> 1), jnp.float32)
    for _ in range(3):
        y = y * (1.5 - 0.5 * x * y * y)
    return y


def _deg_body(dst_hbm, zeros1_hbm, dinv_hbm,
              deg_sh, dst_v, ones_v, dinv_v, dsem):
    cid = lax.axis_index("c")
    sid = lax.axis_index("s")
    base = sid * RPT
    pltpu.sync_copy(zeros1_hbm.at[pl.ds(base, RPT)],
                    deg_sh.at[pl.ds(base, RPT)])
    pltpu.sync_copy(dst_hbm.at[cid, sid], dst_v)
    for i in range(CHUNK // 16):
        ones_v[pl.ds(i * 16, 16)] = jnp.ones((16,), jnp.float32)
    plsc.subcore_barrier()

    def deg_wave(w, carry):
        def fire(j, c):
            pltpu.async_copy(ones_v, deg_sh.at[dst_v.at[w * 40 + j]], dsem,
                             add=True)
            return c
        lax.fori_loop(0, 40, fire, 0)

        def drain(j, c):
            pltpu.make_async_copy(
                ones_v, deg_sh.at[dst_v.at[w * 40 + j]], dsem).wait()
            return c
        lax.fori_loop(0, 40, drain, 0)
        return carry

    lax.fori_loop(0, NCHUNK // 40, deg_wave, 0)
    plsc.subcore_barrier()
    pltpu.sync_copy(deg_sh.at[pl.ds(base, RPT)], dinv_v)

    def newton(k, carry):
        x = dinv_v[pl.ds(k * 16, 16)] + 1.0
        dinv_v[pl.ds(k * 16, 16)] = _rsqrt_newton(x)
        return carry

    lax.fori_loop(0, RPT // 16, newton, 0)
    pltpu.sync_copy(dinv_v, dinv_hbm.at[cid, pl.ds(base, RPT)])


def _sc_body(h_hbm, src_hbm, dst_hbm, batch_hbm, b1_hbm, dinv_hbm,
             zeros2_hbm,
             sum_hbm, cnt_hbm,
             tab_sh, acc_sh, pool_sh, pcnt_sh,
             src_v, dst_v, batch_v, ones_v, b1_v, dinv_v, row_v, acc_v,
             pool_v, pcnt_v, idx_v,
             rows, gsems, ssems, dsem):
    cid = lax.axis_index("c")
    sid = lax.axis_index("s")
    base = sid * RPT

    ld_dinv = pltpu.async_copy(dinv_hbm.at[cid, pl.ds(base, RPT)], dinv_v,
                               gsems[0])
    ld_zero = pltpu.async_copy(zeros2_hbm.at[pl.ds(base, RPT)],
                               acc_sh.at[pl.ds(base, RPT)], gsems[1])
    ld_h = pltpu.async_copy(h_hbm.at[cid, pl.ds(base, RPT)], row_v, gsems[2])
    ld_src = pltpu.async_copy(src_hbm.at[cid, sid], src_v, gsems[3])
    ld_dst = pltpu.async_copy(dst_hbm.at[cid, sid], dst_v, gsems[4])
    ld_bat = pltpu.async_copy(batch_hbm.at[cid, pl.ds(base, RPT)], batch_v,
                              gsems[5])
    ld_b1 = pltpu.async_copy(b1_hbm.at[cid], b1_v, gsems[6])

    @pl.when(sid == 0)
    def _():
        pltpu.async_copy(zeros2_hbm.at[pl.ds(0, CHUNK)], pool_sh,
                         ssems[0]).wait()
        pltpu.async_copy(zeros2_hbm.at[pl.ds(0, CHUNK)], pcnt_sh,
                         ssems[1]).wait()

    for i in range(CHUNK // 16):
        ones_v[pl.ds(i * 16, 16)] = jnp.ones((16,), jnp.float32)
        idx_v[pl.ds(i * 16, 16)] = (
            lax.iota(jnp.int32, 16) + jnp.int32(i * 16))
    zero16 = jnp.zeros((16,), jnp.float32)

    def zero_pools(i, carry):
        pool_v[i, :] = zero16
        pcnt_v[i, :] = zero16
        return carry

    lax.fori_loop(0, CHUNK, zero_pools, 0)
    ld_dinv.wait()
    ld_zero.wait()
    ld_h.wait()
    ld_src.wait()
    ld_dst.wait()
    ld_bat.wait()
    ld_b1.wait()

    def premul(k, carry):
        dv16 = dinv_v[pl.ds(k * 16, 16)]
        for t in range(16):
            i = k * 16 + t
            row_v[i, :] = row_v[i, :] * dv16[t]
        return carry

    lax.fori_loop(0, RPT // 16, premul, 0)
    pltpu.sync_copy(row_v, tab_sh.at[pl.ds(base, RPT)])
    plsc.subcore_barrier()

    def edge_pass():
        for b in range(NRING):
            pltpu.async_copy(tab_sh.at[src_v.at[b]], rows[b], gsems[b])

        def rnd(jj, carry):
            for b in range(NRING):
                j = jj * NRING + b
                pltpu.make_async_copy(tab_sh.at[src_v.at[j]], rows[b],
                                      gsems[b]).wait()
                pltpu.async_copy(rows[b], acc_sh.at[dst_v.at[j]], ssems[b],
                                 add=True)
            for b in range(NRING):
                j2 = (jj + 1) * NRING + b
                pltpu.make_async_copy(rows[b], acc_sh.at[dst_v.at[j2]],
                                      ssems[b]).wait()
                pltpu.async_copy(tab_sh.at[src_v.at[j2]], rows[b], gsems[b])
            return carry

        lax.fori_loop(0, NCHUNK // NRING - 1, rnd, 0)
        jl = NCHUNK - NRING
        for b in range(NRING):
            pltpu.make_async_copy(tab_sh.at[src_v.at[jl + b]], rows[b],
                                  gsems[b]).wait()
            pltpu.sync_copy(rows[b], acc_sh.at[dst_v.at[jl + b]], add=True)

    edge_pass()
    plsc.subcore_barrier()

    pltpu.sync_copy(acc_sh.at[pl.ds(base, RPT)], acc_v)
    pltpu.sync_copy(zeros2_hbm.at[pl.ds(base, RPT)],
                    acc_sh.at[pl.ds(base, RPT)])

    b1vec = b1_v[...]

    def combine1(k, carry):
        dv16 = dinv_v[pl.ds(k * 16, 16)]
        for t in range(16):
            i = k * 16 + t
            r = jnp.maximum((acc_v[i, :] + row_v[i, :]) * dv16[t] + b1vec,
                            0.0)
            row_v[i, :] = r * dv16[t]
        return carry

    lax.fori_loop(0, RPT // 16, combine1, 0)
    pltpu.sync_copy(row_v, tab_sh.at[pl.ds(base, RPT)])
    plsc.subcore_barrier()

    edge_pass()
    plsc.subcore_barrier()

    pltpu.sync_copy(acc_sh.at[pl.ds(base, RPT)], acc_v)
    one16 = jnp.ones((16,), jnp.float32)

    def combine2(k, carry):
        dv16 = dinv_v[pl.ds(k * 16, 16)]
        b16 = batch_v[pl.ds(k * 16, 16)]
        for t in range(16):
            i = k * 16 + t
            bi = b16[t]
            pre = (acc_v[i, :] + row_v[i, :]) * dv16[t]
            pool_v[bi, :] = pool_v[bi, :] + pre
            pcnt_v[bi, :] = pcnt_v[bi, :] + one16
        return carry

    lax.fori_loop(0, RPT // 16, combine2, 0)

    pltpu.sync_copy(pool_v, pool_sh.at[idx_v], add=True)
    pltpu.sync_copy(pcnt_v, pcnt_sh.at[idx_v], add=True)
    plsc.subcore_barrier()

    @pl.when(sid == 0)
    def _():
        pltpu.sync_copy(pool_sh.at[pl.ds(0, G)], sum_hbm.at[cid])
        pltpu.sync_copy(pcnt_sh.at[pl.ds(0, G)], cnt_hbm.at[cid])


@functools.cache
def _sc_kernels():
    mesh = plsc.VectorSubcoreMesh(core_axis_name="c", subcore_axis_name="s",
                                  num_cores=NC, num_subcores=NS)
    deg_kernel = pl.kernel(
        _deg_body,
        out_type=jax.ShapeDtypeStruct((NC, N_PAD), jnp.float32),
        mesh=mesh,
        compiler_params=pltpu.CompilerParams(use_tc_tiling_on_sc=False,
                                             needs_layout_passes=False),
        scratch_types=[
            pltpu.VMEM_SHARED((N_PAD,), jnp.float32),
            pltpu.VMEM((NCHUNK, CHUNK), jnp.int32),
            pltpu.VMEM((CHUNK,), jnp.float32),
            pltpu.VMEM((RPT,), jnp.float32),
            pltpu.SemaphoreType.DMA,
        ],
    )
    main_kernel = pl.kernel(
        _sc_body,
        out_type=[
            jax.ShapeDtypeStruct((NC, G, F1), jnp.float32),
            jax.ShapeDtypeStruct((NC, G, F1), jnp.float32),
        ],
        mesh=mesh,
        compiler_params=pltpu.CompilerParams(use_tc_tiling_on_sc=False,
                                             needs_layout_passes=False),
        scratch_types=[
            pltpu.VMEM_SHARED((N_PAD, F1), jnp.float32),
            pltpu.VMEM_SHARED((N_PAD, F1), jnp.float32),
            pltpu.VMEM_SHARED((CHUNK, F1), jnp.float32),
            pltpu.VMEM_SHARED((CHUNK, F1), jnp.float32),
            pltpu.VMEM((NCHUNK, CHUNK), jnp.int32),
            pltpu.VMEM((NCHUNK, CHUNK), jnp.int32),
            pltpu.VMEM((RPT,), jnp.int32),
            pltpu.VMEM((CHUNK,), jnp.float32),
            pltpu.VMEM((F1,), jnp.float32),
            pltpu.VMEM((RPT,), jnp.float32),
            pltpu.VMEM((RPT, F1), jnp.float32),
            pltpu.VMEM((RPT, F1), jnp.float32),
            pltpu.VMEM((CHUNK, F1), jnp.float32),
            pltpu.VMEM((CHUNK, F1), jnp.float32),
            pltpu.VMEM((CHUNK,), jnp.int32),
            [pltpu.VMEM((CHUNK, F1), jnp.float32)] * NRING,
            [pltpu.SemaphoreType.DMA] * NRING,
            [pltpu.SemaphoreType.DMA] * NRING,
            pltpu.SemaphoreType.DMA,
        ],
    )
    return deg_kernel, main_kernel


def _tca_body(xp_ref, xl_ref, w_ref, h_ref):
    b = pl.program_id(0)
    row = b * BLK + lax.broadcasted_iota(jnp.int32, (BLK, 1), 0)
    mask = row < N
    hp = jnp.dot(xp_ref[...], w_ref[0], preferred_element_type=jnp.float32)
    hl = jnp.dot(xl_ref[...], w_ref[1], preferred_element_type=jnp.float32)
    h_ref[0] = jnp.where(mask, hp, 0.0)
    h_ref[1] = jnp.where(mask, hl, 0.0)


def _tc_a(xp, xl, w1s):
    return pl.pallas_call(
        _tca_body,
        grid=(NB,),
        in_specs=[
            pl.BlockSpec((BLK, D), lambda b: (b, 0)),
            pl.BlockSpec((BLK, D), lambda b: (b, 0)),
            pl.BlockSpec((2, D, F1), lambda b: (0, 0, 0)),
        ],
        out_specs=pl.BlockSpec((2, BLK, F1), lambda b: (0, b, 0)),
        out_shape=jax.ShapeDtypeStruct((2, N_PAD, F1), jnp.float32),
    )(xp, xl, w1s)


def _tcc_body(sum_ref, cnt_ref, w2_ref, b2_ref,
              wa_ref, ba_ref, wo_ref, bo_ref, out_ref):
    def pool(k):
        mean = sum_ref[k] / jnp.maximum(cnt_ref[k], 1.0)
        return jnp.dot(mean, w2_ref[k],
                       preferred_element_type=jnp.float32) + b2_ref[k]

    m = jnp.concatenate([pool(0), pool(1)], axis=1)
    a = jnp.maximum(jnp.dot(m, wa_ref[...],
                            preferred_element_type=jnp.float32)
                    + ba_ref[...], 0.0)
    o = jnp.dot(a, wo_ref[...],
                preferred_element_type=jnp.float32) + bo_ref[...]
    out_ref[...] = jnp.tanh(o)


def _tc_c(sums, cnts, w2s, b2s, wa, ba, wo, bo):
    return pl.pallas_call(
        _tcc_body,
        out_shape=jax.ShapeDtypeStruct((G, 40), jnp.float32),
    )(sums, cnts, w2s, b2s, wa, ba, wo, bo)


def _pad_edges(ei):
    pad = jnp.full((2, E_PAD - E), N, jnp.int32)
    e = jnp.concatenate([ei.astype(jnp.int32), pad], axis=1)
    return e.reshape(2, NS, NCHUNK, CHUNK)


def kernel(protein_x, protein_edge_index, protein_batch,
           ligand_x, ligand_edge_index, ligand_batch,
           W_p_in, b_p_in, W_p_out, b_p_out,
           W_l_in, b_l_in, W_l_out, b_l_out,
           W_a_in, b_a_in, W_a_out, b_a_out):
    ep = _pad_edges(protein_edge_index)
    el = _pad_edges(ligand_edge_index)
    src4 = jnp.stack([ep[0], el[0]])
    dst4 = jnp.stack([ep[1], el[1]])
    batch2 = jnp.stack([
        jnp.pad(protein_batch.astype(jnp.int32), (0, N_PAD - N),
                constant_values=G),
        jnp.pad(ligand_batch.astype(jnp.int32), (0, N_PAD - N),
                constant_values=G),
    ])

    w1s = jnp.stack([W_p_in, W_l_in])
    b1s = jnp.stack([b_p_in, b_l_in])
    w2s = jnp.stack([W_p_out, W_l_out])
    b2s = jnp.stack([b_p_out, b_l_out]).reshape(2, 1, 50)

    zeros1 = jnp.zeros((N_PAD,), jnp.float32)
    zeros2 = jnp.zeros((N_PAD, F1), jnp.float32)

    deg_kernel, main_kernel = _sc_kernels()
    dinv = deg_kernel(dst4, zeros1)
    hs = _tc_a(protein_x, ligand_x, w1s)
    sums, cnts = main_kernel(hs, src4, dst4, batch2, b1s, dinv, zeros2)
    return _tc_c(sums, cnts, w2s, b2s,
                 W_a_in, b_a_in.reshape(1, 60), W_a_out, b_a_out.reshape(1, 40))


# speedup vs baseline: 1.0126x; 1.0126x over previous
import functools

import jax
import jax.numpy as jnp
from jax import lax
from jax.experimental import pallas as pl
from jax.experimental.pallas import tpu as pltpu
from jax.experimental.pallas import tpu_sc as plsc

N = 10000
E = 160000
D = 256
G = 64
F1 = 16

NC = 2
NS = 16
CHUNK = 128
NCHUNK = 80
E_TILE = NCHUNK * CHUNK
E_PAD = E_TILE * NS
N_PAD = 10240
RPT = N_PAD // NS
NB = 10
BLK = N_PAD // NB
NRING = 8


def _rsqrt_newton(x):
    i = plsc.bitcast(x, jnp.int32)
    y = plsc.bitcast(jnp.int32(0x5F3759DF) - (i >> 1), jnp.float32)
    for _ in range(3):
        y = y * (1.5 - 0.5 * x * y * y)
    return y


def _deg_body(dst_hbm, zeros1_hbm, dinv_hbm,
              deg_sh, dst_v, ones_v, dinv_v, dsem):
    cid = lax.axis_index("c")
    sid = lax.axis_index("s")
    base = sid * RPT
    pltpu.sync_copy(zeros1_hbm.at[pl.ds(base, RPT)],
                    deg_sh.at[pl.ds(base, RPT)])
    pltpu.sync_copy(dst_hbm.at[cid, sid], dst_v)
    for i in range(CHUNK // 16):
        ones_v[pl.ds(i * 16, 16)] = jnp.ones((16,), jnp.float32)
    plsc.subcore_barrier()

    def deg_wave(w, carry):
        def fire(j, c):
            pltpu.async_copy(ones_v, deg_sh.at[dst_v.at[w * 40 + j]], dsem,
                             add=True)
            return c
        lax.fori_loop(0, 40, fire, 0)

        def drain(j, c):
            pltpu.make_async_copy(
                ones_v, deg_sh.at[dst_v.at[w * 40 + j]], dsem).wait()
            return c
        lax.fori_loop(0, 40, drain, 0)
        return carry

    lax.fori_loop(0, NCHUNK // 40, deg_wave, 0)
    plsc.subcore_barrier()
    pltpu.sync_copy(deg_sh.at[pl.ds(base, RPT)], dinv_v)

    def newton(k, carry):
        x = dinv_v[pl.ds(k * 16, 16)] + 1.0
        dinv_v[pl.ds(k * 16, 16)] = _rsqrt_newton(x)
        return carry

    lax.fori_loop(0, RPT // 16, newton, 0)
    pltpu.sync_copy(dinv_v, dinv_hbm.at[cid, pl.ds(base, RPT)])


def _sc_body(h_hbm, src_hbm, dst_hbm, batch_hbm, b1_hbm, dinv_hbm,
             zeros2_hbm,
             sum_hbm, cnt_hbm,
             tab_sh, acc_sh, pool_sh, pcnt_sh,
             src_v, dst_v, batch_v, ones_v, b1_v, dinv_v, row_v, acc_v,
             pool_v, pcnt_v, idx_v,
             rows, gsems, ssems, dsem):
    cid = lax.axis_index("c")
    sid = lax.axis_index("s")
    base = sid * RPT

    ld_dinv = pltpu.async_copy(dinv_hbm.at[cid, pl.ds(base, RPT)], dinv_v,
                               gsems[0])
    ld_zero = pltpu.async_copy(zeros2_hbm.at[pl.ds(base, RPT)],
                               acc_sh.at[pl.ds(base, RPT)], gsems[1])
    ld_h = pltpu.async_copy(h_hbm.at[cid, pl.ds(base, RPT)], row_v, gsems[2])
    ld_src = pltpu.async_copy(src_hbm.at[cid, sid], src_v, gsems[3])
    ld_dst = pltpu.async_copy(dst_hbm.at[cid, sid], dst_v, gsems[4])
    ld_bat = pltpu.async_copy(batch_hbm.at[cid, pl.ds(base, RPT)], batch_v,
                              gsems[5])
    ld_b1 = pltpu.async_copy(b1_hbm.at[cid], b1_v, gsems[6])

    @pl.when(sid == 0)
    def _():
        pltpu.async_copy(zeros2_hbm.at[pl.ds(0, CHUNK)], pool_sh,
                         ssems[0]).wait()
        pltpu.async_copy(zeros2_hbm.at[pl.ds(0, CHUNK)], pcnt_sh,
                         ssems[1]).wait()

    for i in range(CHUNK // 16):
        ones_v[pl.ds(i * 16, 16)] = jnp.ones((16,), jnp.float32)
        idx_v[pl.ds(i * 16, 16)] = (
            lax.iota(jnp.int32, 16) + jnp.int32(i * 16))
    zero16 = jnp.zeros((16,), jnp.float32)

    def zero_pools(i, carry):
        pool_v[i, :] = zero16
        pcnt_v[i, :] = zero16
        return carry

    lax.fori_loop(0, CHUNK, zero_pools, 0)
    ld_dinv.wait()
    ld_zero.wait()
    ld_h.wait()
    ld_src.wait()
    ld_dst.wait()
    ld_bat.wait()
    ld_b1.wait()

    def premul(k, carry):
        dv16 = dinv_v[pl.ds(k * 16, 16)]
        for t in range(16):
            i = k * 16 + t
            row_v[i, :] = row_v[i, :] * dv16[t]
        return carry

    lax.fori_loop(0, RPT // 16, premul, 0)
    pltpu.sync_copy(row_v, tab_sh.at[pl.ds(base, RPT)])
    plsc.subcore_barrier()

    def edge_pass():
        for b in range(NRING):
            pltpu.async_copy(tab_sh.at[src_v.at[b]], rows[b], gsems[b])

        def rnd(jj, carry):
            for b in range(NRING):
                j = jj * NRING + b
                pltpu.make_async_copy(tab_sh.at[src_v.at[j]], rows[b],
                                      gsems[b]).wait()
                pltpu.async_copy(rows[b], acc_sh.at[dst_v.at[j]], ssems[b],
                                 add=True)
            for b in range(NRING):
                j2 = (jj + 1) * NRING + b
                pltpu.make_async_copy(rows[b], acc_sh.at[dst_v.at[j2]],
                                      ssems[b]).wait()
                pltpu.async_copy(tab_sh.at[src_v.at[j2]], rows[b], gsems[b])
            return carry

        lax.fori_loop(0, NCHUNK // NRING - 1, rnd, 0)
        jl = NCHUNK - NRING
        for b in range(NRING):
            pltpu.make_async_copy(tab_sh.at[src_v.at[jl + b]], rows[b],
                                  gsems[b]).wait()
            pltpu.sync_copy(rows[b], acc_sh.at[dst_v.at[jl + b]], add=True)

    edge_pass()
    plsc.subcore_barrier()

    pltpu.sync_copy(acc_sh.at[pl.ds(base, RPT)], acc_v)
    pltpu.sync_copy(zeros2_hbm.at[pl.ds(base, RPT)],
                    acc_sh.at[pl.ds(base, RPT)])

    b1vec = b1_v[...]

    def combine1(k, carry):
        dv16 = dinv_v[pl.ds(k * 16, 16)]
        for t in range(16):
            i = k * 16 + t
            r = jnp.maximum((acc_v[i, :] + row_v[i, :]) * dv16[t] + b1vec,
                            0.0)
            row_v[i, :] = r * dv16[t]
        return carry

    lax.fori_loop(0, RPT // 16, combine1, 0)
    pltpu.sync_copy(row_v, tab_sh.at[pl.ds(base, RPT)])
    plsc.subcore_barrier()

    edge_pass()
    plsc.subcore_barrier()

    pltpu.sync_copy(acc_sh.at[pl.ds(base, RPT)], acc_v)
    one16 = jnp.ones((16,), jnp.float32)

    def combine2(k, carry):
        dv16 = dinv_v[pl.ds(k * 16, 16)]
        b16 = batch_v[pl.ds(k * 16, 16)]
        for t in range(16):
            i = k * 16 + t
            bi = b16[t]
            pre = (acc_v[i, :] + row_v[i, :]) * dv16[t]
            pool_v[bi, :] = pool_v[bi, :] + pre
            pcnt_v[bi, :] = pcnt_v[bi, :] + one16
        return carry

    lax.fori_loop(0, RPT // 16, combine2, 0)

    pltpu.sync_copy(pool_v, pool_sh.at[idx_v], add=True)
    pltpu.sync_copy(pcnt_v, pcnt_sh.at[idx_v], add=True)
    plsc.subcore_barrier()

    @pl.when(sid == 0)
    def _():
        pltpu.sync_copy(pool_sh.at[pl.ds(0, G)], sum_hbm.at[cid])
        pltpu.sync_copy(pcnt_sh.at[pl.ds(0, G)], cnt_hbm.at[cid])


@functools.cache
def _sc_kernels():
    mesh = plsc.VectorSubcoreMesh(core_axis_name="c", subcore_axis_name="s",
                                  num_cores=NC, num_subcores=NS)
    deg_kernel = pl.kernel(
        _deg_body,
        out_type=jax.ShapeDtypeStruct((NC, N_PAD), jnp.float32),
        mesh=mesh,
        compiler_params=pltpu.CompilerParams(use_tc_tiling_on_sc=False,
                                             needs_layout_passes=False),
        scratch_types=[
            pltpu.VMEM_SHARED((N_PAD,), jnp.float32),
            pltpu.VMEM((NCHUNK, CHUNK), jnp.int32),
            pltpu.VMEM((CHUNK,), jnp.float32),
            pltpu.VMEM((RPT,), jnp.float32),
            pltpu.SemaphoreType.DMA,
        ],
    )
    main_kernel = pl.kernel(
        _sc_body,
        out_type=[
            jax.ShapeDtypeStruct((NC, G, F1), jnp.float32),
            jax.ShapeDtypeStruct((NC, G, F1), jnp.float32),
        ],
        mesh=mesh,
        compiler_params=pltpu.CompilerParams(use_tc_tiling_on_sc=False,
                                             needs_layout_passes=False),
        scratch_types=[
            pltpu.VMEM_SHARED((N_PAD, F1), jnp.float32),
            pltpu.VMEM_SHARED((N_PAD, F1), jnp.float32),
            pltpu.VMEM_SHARED((CHUNK, F1), jnp.float32),
            pltpu.VMEM_SHARED((CHUNK, F1), jnp.float32),
            pltpu.VMEM((NCHUNK, CHUNK), jnp.int32),
            pltpu.VMEM((NCHUNK, CHUNK), jnp.int32),
            pltpu.VMEM((RPT,), jnp.int32),
            pltpu.VMEM((CHUNK,), jnp.float32),
            pltpu.VMEM((F1,), jnp.float32),
            pltpu.VMEM((RPT,), jnp.float32),
            pltpu.VMEM((RPT, F1), jnp.float32),
            pltpu.VMEM((RPT, F1), jnp.float32),
            pltpu.VMEM((CHUNK, F1), jnp.float32),
            pltpu.VMEM((CHUNK, F1), jnp.float32),
            pltpu.VMEM((CHUNK,), jnp.int32),
            [pltpu.VMEM((CHUNK, F1), jnp.float32)] * NRING,
            [pltpu.SemaphoreType.DMA] * NRING,
            [pltpu.SemaphoreType.DMA] * NRING,
            pltpu.SemaphoreType.DMA,
        ],
    )
    return deg_kernel, main_kernel


BLKA = 2048
NBA = N_PAD // BLKA


def _tca_body(xp_ref, xl_ref, w_ref, h_ref):
    b = pl.program_id(0)
    row = b * BLKA + lax.broadcasted_iota(jnp.int32, (BLKA, 1), 0)
    mask = row < N
    hp = jnp.dot(xp_ref[...], w_ref[0], preferred_element_type=jnp.float32)
    hl = jnp.dot(xl_ref[...], w_ref[1], preferred_element_type=jnp.float32)
    h_ref[0] = jnp.where(mask, hp, 0.0)
    h_ref[1] = jnp.where(mask, hl, 0.0)


def _tc_a(xp, xl, w1s):
    return pl.pallas_call(
        _tca_body,
        grid=(NBA,),
        in_specs=[
            pl.BlockSpec((BLKA, D), lambda b: (b, 0)),
            pl.BlockSpec((BLKA, D), lambda b: (b, 0)),
            pl.BlockSpec((2, D, F1), lambda b: (0, 0, 0)),
        ],
        out_specs=pl.BlockSpec((2, BLKA, F1), lambda b: (0, b, 0)),
        out_shape=jax.ShapeDtypeStruct((2, N_PAD, F1), jnp.float32),
    )(xp, xl, w1s)


def _tcc_body(sum_ref, cnt_ref, w2_ref, b2_ref,
              wa_ref, ba_ref, wo_ref, bo_ref, out_ref):
    def pool(k):
        mean = sum_ref[k] / jnp.maximum(cnt_ref[k], 1.0)
        return jnp.dot(mean, w2_ref[k],
                       preferred_element_type=jnp.float32) + b2_ref[k]

    m = jnp.concatenate([pool(0), pool(1)], axis=1)
    a = jnp.maximum(jnp.dot(m, wa_ref[...],
                            preferred_element_type=jnp.float32)
                    + ba_ref[...], 0.0)
    o = jnp.dot(a, wo_ref[...],
                preferred_element_type=jnp.float32) + bo_ref[...]
    out_ref[...] = jnp.tanh(o)


def _tc_c(sums, cnts, w2s, b2s, wa, ba, wo, bo):
    return pl.pallas_call(
        _tcc_body,
        out_shape=jax.ShapeDtypeStruct((G, 40), jnp.float32),
    )(sums, cnts, w2s, b2s, wa, ba, wo, bo)


def _pad_edges(ei):
    pad = jnp.full((2, E_PAD - E), N, jnp.int32)
    e = jnp.concatenate([ei.astype(jnp.int32), pad], axis=1)
    return e.reshape(2, NS, NCHUNK, CHUNK)


def kernel(protein_x, protein_edge_index, protein_batch,
           ligand_x, ligand_edge_index, ligand_batch,
           W_p_in, b_p_in, W_p_out, b_p_out,
           W_l_in, b_l_in, W_l_out, b_l_out,
           W_a_in, b_a_in, W_a_out, b_a_out):
    ep = _pad_edges(protein_edge_index)
    el = _pad_edges(ligand_edge_index)
    src4 = jnp.stack([ep[0], el[0]])
    dst4 = jnp.stack([ep[1], el[1]])
    batch2 = jnp.stack([
        jnp.pad(protein_batch.astype(jnp.int32), (0, N_PAD - N),
                constant_values=G),
        jnp.pad(ligand_batch.astype(jnp.int32), (0, N_PAD - N),
                constant_values=G),
    ])

    w1s = jnp.stack([W_p_in, W_l_in])
    b1s = jnp.stack([b_p_in, b_l_in])
    w2s = jnp.stack([W_p_out, W_l_out])
    b2s = jnp.stack([b_p_out, b_l_out]).reshape(2, 1, 50)

    zeros1 = jnp.zeros((N_PAD,), jnp.float32)
    zeros2 = jnp.zeros((N_PAD, F1), jnp.float32)

    deg_kernel, main_kernel = _sc_kernels()
    dinv = deg_kernel(dst4, zeros1)
    hs = _tc_a(protein_x, ligand_x, w1s)
    sums, cnts = main_kernel(hs, src4, dst4, batch2, b1s, dinv, zeros2)
    return _tc_c(sums, cnts, w2s, b2s,
                 W_a_in, b_a_in.reshape(1, 60), W_a_out, b_a_out.reshape(1, 40))
